# baseline (speedup 1.0000x reference)
import functools

import jax
import jax.numpy as jnp
from jax import lax
from jax.experimental import pallas as pl
from jax.experimental.pallas import tpu as pltpu
from jax.experimental.pallas import tpu_sc as plsc

_B = 16384
_K = 16
_NC = 2
_NS = 16
_NW = _NC * _NS
_BPW = _B // _NW
_L = 16
_SEG = 128
_NWAVE = _BPW // _L


def _mf_body(wt_hbm, ht_hbm, uidx_hbm, iidx_hbm, out_hbm,
             uidx_v, iidx_v, ublk, vblk, out_v, sem):
    wid = lax.axis_index("s") * _NC + lax.axis_index("c")
    lane = lax.iota(jnp.int32, _L)

    pltpu.sync_copy(uidx_hbm.at[wid], uidx_v)
    pltpu.sync_copy(iidx_hbm.at[wid], iidx_v)

    def _wave(w, carry):
        row = w >> 3
        col0 = (w & 7) * _L
        uvec = uidx_v[row, pl.ds(col0, _L)]
        ivec = iidx_v[row, pl.ds(col0, _L)]
        copies = []
        for t in range(_L):
            us = jnp.sum(jnp.where(lane == t, uvec, 0))
            vs = jnp.sum(jnp.where(lane == t, ivec, 0))
            uoff = pl.multiple_of((us >> 7) * _SEG, _SEG)
            voff = pl.multiple_of((vs >> 7) * _SEG, _SEG)
            copies.append(pltpu.async_copy(
                wt_hbm.at[:, pl.ds(uoff, _SEG)], ublk.at[t], sem))
            copies.append(pltpu.async_copy(
                ht_hbm.at[:, pl.ds(voff, _SEG)], vblk.at[t], sem))
        for c in copies:
            c.wait()

        ucol = uvec & (_SEG - 1)
        icol = ivec & (_SEG - 1)
        acc = jnp.zeros((_L,), jnp.float32)
        for k in range(_K):
            kv = jnp.full((_L,), k, jnp.int32)
            u = plsc.load_gather(ublk, [lane, kv, ucol])
            v = plsc.load_gather(vblk, [lane, kv, icol])
            acc = acc + u * v
        out_v[pl.ds(w * _L, _L)] = 1.0 / (1.0 + jnp.exp(-acc))
        return carry

    lax.fori_loop(0, _NWAVE, _wave, 0)

    pltpu.sync_copy(out_v, out_hbm.at[pl.ds(wid * _BPW, _BPW)])


_mf_call = functools.partial(
    pl.kernel,
    out_type=jax.ShapeDtypeStruct((_B,), jnp.float32),
    mesh=plsc.VectorSubcoreMesh(core_axis_name="c", subcore_axis_name="s"),
    scratch_types=[
        pltpu.VMEM((_BPW // 128, 128), jnp.int32),
        pltpu.VMEM((_BPW // 128, 128), jnp.int32),
        pltpu.VMEM((_L, _K, _SEG), jnp.float32),
        pltpu.VMEM((_L, _K, _SEG), jnp.float32),
        pltpu.VMEM((_BPW,), jnp.float32),
        pltpu.SemaphoreType.DMA,
    ],
    compiler_params=pltpu.CompilerParams(
        needs_layout_passes=False, use_tc_tiling_on_sc=True),
)(_mf_body)


def kernel(x, W, H):
    wt = W.T
    ht = H.T
    shape = (_NW, _BPW // 128, 128)
    uidx = x[:, 0].reshape(shape)
    iidx = x[:, 1].reshape(shape)
    return _mf_call(wt, ht, uidx, iidx)
